# baseline (speedup 1.0000x reference)
import functools

import jax
import jax.numpy as jnp
from jax import lax
from jax.experimental import pallas as pl
from jax.experimental.pallas import tpu as pltpu
from jax.experimental.pallas import tpu_sc as plsc

N = 10000
D = 128
H = 16
NC = 2
NS = 16
NW = NC * NS
RPS = 625
NPAD = NS * RPS
DH = D // (2 * NC)

CH_A = 200
CPT_A = 100
NB_A = 4

CH_B = 400
CPT_B = 25
NB_B = 5


def _sc_agg128(x, src_t, dst_t, p):
    mesh = plsc.VectorSubcoreMesh(core_axis_name="c", subcore_axis_name="s")

    @functools.partial(
        pl.kernel,
        out_type=jax.ShapeDtypeStruct((NC, NPAD, DH), jnp.float32),
        mesh=mesh,
        scratch_types=[
            pltpu.VMEM((CPT_A, CH_A), jnp.int32),
            pltpu.VMEM((CPT_A, CH_A), jnp.int32),
            pltpu.VMEM((NB_A, CH_A, DH), jnp.float32),
            pltpu.VMEM_SHARED((NPAD, DH), jnp.float32),
            pltpu.SemaphoreType.DMA,
        ],
        compiler_params=pltpu.CompilerParams(use_tc_tiling_on_sc=False),
    )
    def k(x_hbm, src_hbm, dst_hbm, out_hbm, src_v, dst_v, rows_v,
          agg_sh, sem):
        cid = lax.axis_index("c")
        sid = lax.axis_index("s")
        x_half = x_hbm.at[2 * p + cid]

        pltpu.sync_copy(src_hbm.at[sid], src_v)
        pltpu.sync_copy(dst_hbm.at[sid], dst_v)

        def zrow(i, carry):
            for kk in range(DH // 16):
                rows_v[0, i, pl.ds(kk * 16, 16)] = jnp.zeros((16,), jnp.float32)
            return carry

        lax.fori_loop(0, CH_A, zrow, 0)
        base = sid * RPS
        for off, nb in ((0, 200), (200, 200), (400, 200), (600, 25)):
            pltpu.sync_copy(rows_v.at[0, pl.ds(0, nb)],
                            agg_sh.at[pl.ds(base + off, nb)])
        plsc.subcore_barrier()

        for b in range(NB_A):
            pltpu.async_copy(x_half.at[src_v.at[b]], rows_v.at[b], sem)

        def group(g, carry):
            for b in range(NB_A):
                j = g * NB_A + b
                pltpu.make_async_copy(x_half.at[src_v.at[j]],
                                      rows_v.at[b], sem).wait()
                pltpu.sync_copy(rows_v.at[b], agg_sh.at[dst_v.at[j]], add=True)
                pltpu.async_copy(x_half.at[src_v.at[j + NB_A]], rows_v.at[b], sem)
            return carry

        lax.fori_loop(0, CPT_A // NB_A - 1, group, 0)
        for b in range(NB_A):
            j = CPT_A - NB_A + b
            pltpu.make_async_copy(x_half.at[src_v.at[j]],
                                  rows_v.at[b], sem).wait()
            pltpu.sync_copy(rows_v.at[b], agg_sh.at[dst_v.at[j]], add=True)
        plsc.subcore_barrier()
        pltpu.sync_copy(agg_sh.at[pl.ds(sid * RPS, RPS)],
                        out_hbm.at[cid, pl.ds(sid * RPS, RPS)])

    return k(x, src_t, dst_t)


def _sc_agg16(y, src_t, dst_t):
    mesh = plsc.VectorSubcoreMesh(core_axis_name="c", subcore_axis_name="s")

    @functools.partial(
        pl.kernel,
        out_type=jax.ShapeDtypeStruct((NC, NPAD, H), jnp.float32),
        mesh=mesh,
        scratch_types=[
            pltpu.VMEM((CPT_B, CH_B), jnp.int32),
            pltpu.VMEM((CPT_B, CH_B), jnp.int32),
            pltpu.VMEM((NB_B, CH_B, H), jnp.float32),
            pltpu.VMEM((RPS, H), jnp.float32),
            pltpu.VMEM_SHARED((NPAD, H), jnp.float32),
            pltpu.VMEM_SHARED((NPAD, H), jnp.float32),
            pltpu.SemaphoreType.DMA,
        ],
        compiler_params=pltpu.CompilerParams(use_tc_tiling_on_sc=False),
    )
    def k(y_hbm, src_hbm, dst_hbm, out_hbm, src_v, dst_v, rows_v, zero_v,
          agg_sh, y_sh, sem):
        cid = lax.axis_index("c")
        sid = lax.axis_index("s")
        wid = cid * NS + sid

        pltpu.sync_copy(src_hbm.at[wid], src_v)
        pltpu.sync_copy(dst_hbm.at[wid], dst_v)
        pltpu.sync_copy(y_hbm.at[pl.ds(sid * RPS, RPS)],
                        y_sh.at[pl.ds(sid * RPS, RPS)])

        def zrow(i, carry):
            zero_v[i, :] = jnp.zeros((H,), jnp.float32)
            return carry

        lax.fori_loop(0, RPS, zrow, 0)
        pltpu.sync_copy(zero_v, agg_sh.at[pl.ds(sid * RPS, RPS)])
        plsc.subcore_barrier()

        for b in range(NB_B):
            pltpu.async_copy(y_sh.at[src_v.at[b]], rows_v.at[b], sem)

        def group(g, carry):
            for b in range(NB_B):
                j = g * NB_B + b
                pltpu.make_async_copy(y_sh.at[src_v.at[j]],
                                      rows_v.at[b], sem).wait()
                pltpu.sync_copy(rows_v.at[b], agg_sh.at[dst_v.at[j]], add=True)
                pltpu.async_copy(y_sh.at[src_v.at[j + NB_B]], rows_v.at[b], sem)
            return carry

        lax.fori_loop(0, CPT_B // NB_B - 1, group, 0)
        for b in range(NB_B):
            j = CPT_B - NB_B + b
            pltpu.make_async_copy(y_sh.at[src_v.at[j]],
                                  rows_v.at[b], sem).wait()
            pltpu.sync_copy(rows_v.at[b], agg_sh.at[dst_v.at[j]], add=True)
        plsc.subcore_barrier()
        pltpu.sync_copy(agg_sh.at[pl.ds(sid * RPS, RPS)],
                        out_hbm.at[cid, pl.ds(sid * RPS, RPS)])

    return k(y, src_t, dst_t)


def _tc_layer1(agga, aggb, x, w_rel, b_rel, w_root):
    def body(agga_ref, aggb_ref, x_ref, wrel_ref, b_ref, wroot_ref, o_ref):
        a = jnp.concatenate([agga_ref[0], agga_ref[1],
                             aggb_ref[0], aggb_ref[1]], axis=1)
        o_ref[...] = jax.nn.relu(a @ wrel_ref[...] + b_ref[...]
                                 + x_ref[...] @ wroot_ref[...])

    return pl.pallas_call(
        body,
        out_shape=jax.ShapeDtypeStruct((NPAD, H), jnp.float32),
    )(agga, aggb, x, w_rel, b_rel, w_root)


def _tc_layer2_head(agg, h1p, w_rel, b_rel, w_root, wfc, bfc):
    def body(agg_ref, h1_ref, wrel_ref, b_ref, wroot_ref, wfc_ref, bfc_ref,
             o_ref):
        a = agg_ref[0] + agg_ref[1]
        h2 = jax.nn.relu(a @ wrel_ref[...] + b_ref[...]
                         + h1_ref[...] @ wroot_ref[...])
        o_ref[...] = h2 @ wfc_ref[...] + bfc_ref[...]

    return pl.pallas_call(
        body,
        out_shape=jax.ShapeDtypeStruct((N, 1), jnp.float32),
    )(agg, h1p, w_rel, b_rel, w_root, wfc, bfc)


def kernel(x, edge_index, W1_rel, b1, W1_root, W2_rel, b2, W2_root, W_fc, b_fc):
    src_a = edge_index[0].reshape(NS, CPT_A, CH_A)
    dst_a = edge_index[1].reshape(NS, CPT_A, CH_A)
    src_b = edge_index[0].reshape(NW, CPT_B, CH_B)
    dst_b = edge_index[1].reshape(NW, CPT_B, CH_B)

    xsplit = x.reshape(N, 2 * NC, DH).swapaxes(0, 1)
    agg1a = _sc_agg128(xsplit, src_a, dst_a, 0)
    agg1b = _sc_agg128(xsplit, src_a, dst_a, 1)
    h1p = _tc_layer1(agg1a, agg1b, x, W1_rel, b1.reshape(1, H), W1_root)
    agg2 = _sc_agg16(h1p, src_b, dst_b)
    out = _tc_layer2_head(agg2, h1p, W2_rel, b2.reshape(1, H), W2_root,
                          W_fc, b_fc.reshape(1, 1))
    return out
